# baseline (speedup 1.0000x reference)
import functools

import jax
import jax.numpy as jnp
from jax import lax
from jax.experimental import pallas as pl
from jax.experimental.pallas import tpu as pltpu
from jax.experimental.pallas import tpu_sc as plsc

_VOCAB = 1000000
_EMBED = 64
_BATCH = 16384
_HIST = 50

_NC = 2
_NS = 16
_L = 16
_NW = _NC * _NS
_NB = _BATCH // _NW
_RING = 4
_DC = _EMBED // _L
_NG = (_HIST + _L - 1) // _L
_HPAD = 56


def _vperm(vec, idx):
    dn = lax.GatherDimensionNumbers(
        offset_dims=(), collapsed_slice_dims=(0,), start_index_map=(0,))
    return lax.gather(vec, idx[:, None], dn, (1,),
                      mode=lax.GatherScatterMode.PROMISE_IN_BOUNDS)


_mesh = plsc.VectorSubcoreMesh(
    core_axis_name="c", subcore_axis_name="s",
    num_cores=_NC, num_subcores=_NS)


@functools.partial(
    pl.kernel,
    out_type=jax.ShapeDtypeStruct((_BATCH, 1, _HIST), jnp.float32),
    mesh=_mesh,
    scratch_types=[
        pltpu.VMEM((_NB, _HPAD), jnp.int32),
        pltpu.VMEM((4, 128), jnp.int32),
        pltpu.VMEM((_NB, _EMBED), jnp.float32),
        pltpu.VMEM((_RING, _HPAD, _EMBED), jnp.float32),
        pltpu.VMEM((_NB, 1, _HIST), jnp.float32),
        pltpu.SemaphoreType.DMA,
        pltpu.SemaphoreType.DMA,
        pltpu.SemaphoreType.DMA,
        pltpu.SemaphoreType.DMA,
        pltpu.SemaphoreType.DMA,
    ],
    compiler_params=pltpu.CompilerParams(
        needs_layout_passes=False, use_tc_tiling_on_sc=False),
)
def _w2v_sc(ctx_idx_hbm, cen_idx_hbm, ctx_tbl, cen_tbl, out_hbm,
            xidx, cidx, cen_all, ctx_ring, out_v,
            cen_sem, sem0, sem1, sem2, sem3):
    cid = lax.axis_index("c")
    sid = lax.axis_index("s")
    wid = sid * _NC + cid
    sems = [sem0, sem1, sem2, sem3]
    iota = lax.iota(jnp.int32, _L)
    zeros16 = jnp.zeros((_L,), jnp.int32)

    pltpu.sync_copy(ctx_idx_hbm.at[pl.ds(wid * _NB, _NB), pl.ds(0, _HPAD)],
                    xidx)
    pltpu.sync_copy(cen_idx_hbm.at[pl.ds(wid * 4, 4)], cidx)

    for j in range(4):
        pltpu.async_copy(cen_tbl.at[cidx.at[j]],
                         cen_all.at[pl.ds(j * 128, 128)], cen_sem)
    for j in range(4):
        pltpu.make_async_copy(cen_tbl.at[cidx.at[j]],
                              cen_all.at[pl.ds(j * 128, 128)], cen_sem).wait()

    def fire(b, slot):
        pltpu.async_copy(ctx_tbl.at[xidx.at[b]], ctx_ring.at[slot], sems[slot])

    def drain(b, slot):
        pltpu.make_async_copy(ctx_tbl.at[xidx.at[b]], ctx_ring.at[slot],
                              sems[slot]).wait()

    rot = [(iota + r) & (_L - 1) for r in range(_L)]
    rows = []
    for g in range(_NG):
        r = g * _L + iota
        rows.append(jnp.minimum(r, _HIST - 1) if g == _NG - 1 else r)
    tail_mask = (( _NG - 1) * _L + iota) < _HIST

    def compute(b, slot):
        def dbody(dc, accs):
            accs = list(accs)
            cen = cen_all[b, pl.ds(dc * _L, _L)]
            base = dc * _L
            for r in range(_L):
                dvec = rot[r] + base
                crot = _vperm(cen, rot[r])
                for g in range(_NG):
                    col = plsc.load_gather(ctx_ring.at[slot], [rows[g], dvec])
                    accs[g] = accs[g] + col * crot
            return tuple(accs)

        accs = lax.fori_loop(
            0, _DC, dbody,
            tuple(jnp.zeros((_L,), jnp.float32) for _ in range(_NG)))
        bvec = jnp.full((_L,), b, jnp.int32)
        for g in range(_NG):
            lvec = g * _L + iota
            if g < _NG - 1:
                plsc.store_scatter(out_v, [bvec, zeros16, lvec], accs[g])
            else:
                plsc.store_scatter(
                    out_v, [bvec, zeros16, jnp.minimum(lvec, _HIST - 1)],
                    accs[g], mask=tail_mask)

    for s in range(_RING):
        fire(s, s)

    def body(i, carry):
        b0 = i * _RING
        for s in range(_RING):
            b = b0 + s
            drain(b, s)
            compute(b, s)

            @pl.when(b + _RING < _NB)
            def _():
                fire(b + _RING, s)
        return carry

    lax.fori_loop(0, _NB // _RING, body, jnp.int32(0))

    pltpu.sync_copy(out_v, out_hbm.at[pl.ds(wid * _NB, _NB)])


def kernel(center, contexts_and_negatives, context_table, center_table):
    ctx_idx = jnp.pad(contexts_and_negatives, ((0, 0), (0, 128 - _HIST)))
    cen_idx = center.reshape(_BATCH // 128, 128)
    return _w2v_sc(ctx_idx, cen_idx, context_table, center_table)


# speedup vs baseline: 2.2155x; 2.2155x over previous
import functools

import jax
import jax.numpy as jnp
from jax import lax
from jax.experimental import pallas as pl
from jax.experimental.pallas import tpu as pltpu
from jax.experimental.pallas import tpu_sc as plsc

_VOCAB = 1000000
_EMBED = 64
_BATCH = 16384
_HIST = 50

_NC = 2
_NS = 16
_L = 16
_NW = _NC * _NS
_NB = _BATCH // _NW
_RING = 4
_DC = _EMBED // _L
_NG = (_HIST + _L - 1) // _L
_HPAD = 56


def _vperm(vec, idx):
    dn = lax.GatherDimensionNumbers(
        offset_dims=(), collapsed_slice_dims=(0,), start_index_map=(0,))
    return lax.gather(vec, idx[:, None], dn, (1,),
                      mode=lax.GatherScatterMode.PROMISE_IN_BOUNDS)


_mesh = plsc.VectorSubcoreMesh(
    core_axis_name="c", subcore_axis_name="s",
    num_cores=_NC, num_subcores=_NS)


@functools.partial(
    pl.kernel,
    out_type=jax.ShapeDtypeStruct((_BATCH, 1, _HIST), jnp.float32),
    mesh=_mesh,
    scratch_types=[
        pltpu.VMEM((_NB, _HPAD), jnp.int32),
        pltpu.VMEM((4, 128), jnp.int32),
        pltpu.VMEM((_NB, _EMBED), jnp.float32),
        pltpu.VMEM((_RING, _HPAD, _EMBED), jnp.float32),
        pltpu.VMEM((_NB, 1, _HIST), jnp.float32),
        pltpu.SemaphoreType.DMA,
        pltpu.SemaphoreType.DMA,
        pltpu.SemaphoreType.DMA,
        pltpu.SemaphoreType.DMA,
        pltpu.SemaphoreType.DMA,
    ],
    compiler_params=pltpu.CompilerParams(
        needs_layout_passes=False, use_tc_tiling_on_sc=False),
)
def _w2v_sc(ctx_idx_hbm, cen_idx_hbm, ctx_tbl, cen_tbl, out_hbm,
            xidx, cidx, cen_all, ctx_ring, out_v,
            cen_sem, sem0, sem1, sem2, sem3):
    cid = lax.axis_index("c")
    sid = lax.axis_index("s")
    wid = sid * _NC + cid
    sems = [sem0, sem1, sem2, sem3]
    iota = lax.iota(jnp.int32, _L)
    zeros16 = jnp.zeros((_L,), jnp.int32)

    pltpu.sync_copy(ctx_idx_hbm.at[pl.ds(wid * _NB, _NB), pl.ds(0, _HPAD)],
                    xidx)
    pltpu.sync_copy(cen_idx_hbm.at[pl.ds(wid * 4, 4)], cidx)

    for j in range(4):
        pltpu.async_copy(cen_tbl.at[cidx.at[j]],
                         cen_all.at[pl.ds(j * 128, 128)], cen_sem)
    for j in range(4):
        pltpu.make_async_copy(cen_tbl.at[cidx.at[j]],
                              cen_all.at[pl.ds(j * 128, 128)], cen_sem).wait()

    def fire(b, slot):
        pltpu.async_copy(ctx_tbl.at[xidx.at[b]], ctx_ring.at[slot], sems[slot])

    def drain(b, slot):
        pltpu.make_async_copy(ctx_tbl.at[xidx.at[b]], ctx_ring.at[slot],
                              sems[slot]).wait()

    rot = [(iota + r) & (_L - 1) for r in range(_L)]
    rows = []
    for g in range(_NG):
        r = g * _L + iota
        rows.append(jnp.minimum(r, _HIST - 1) if g == _NG - 1 else r)
    tail_mask = (( _NG - 1) * _L + iota) < _HIST

    def compute(b, slot):
        def dbody(dc, accs):
            accs = list(accs)
            cen = cen_all[b, pl.ds(dc * _L, _L)]
            base = dc * _L
            for r in range(_L):
                dvec = rot[r] + base
                crot = _vperm(cen, rot[r])
                for g in range(_NG):
                    col = plsc.load_gather(ctx_ring.at[slot], [rows[g], dvec])
                    accs[g] = accs[g] + col * crot
            return tuple(accs)

        accs = lax.fori_loop(
            0, _DC, dbody,
            tuple(jnp.zeros((_L,), jnp.float32) for _ in range(_NG)))
        bvec = jnp.full((_L,), b, jnp.int32)
        for g in range(_NG):
            lvec = g * _L + iota
            if g < _NG - 1:
                plsc.store_scatter(out_v, [bvec, zeros16, lvec], accs[g])
            else:
                plsc.store_scatter(
                    out_v, [bvec, zeros16, jnp.minimum(lvec, _HIST - 1)],
                    accs[g], mask=tail_mask)

    for s in range(_RING):
        fire(s, s)

    def body(i, carry):
        b0 = i * _RING
        for s in range(_RING):
            b = b0 + s
            drain(b, s)
            compute(b, s)

            @pl.when(b + _RING < _NB)
            def _():
                fire(b + _RING, s)
        return carry

    lax.fori_loop(0, _NB // _RING, body, jnp.int32(0))

    pltpu.sync_copy(out_v, out_hbm.at[pl.ds(wid * _NB, _NB)])


def kernel(center, contexts_and_negatives, context_table, center_table):
    ctx_idx = jnp.pad(
        jnp.concatenate(
            [contexts_and_negatives,
             contexts_and_negatives[:, :_HPAD - _HIST]], axis=1),
        ((0, 0), (0, 128 - _HPAD)))
    cen_idx = center.reshape(_BATCH // 128, 128)
    return _w2v_sc(ctx_idx, cen_idx, context_table, center_table)
